# baseline (speedup 1.0000x reference)
import functools

import jax
import jax.numpy as jnp
from jax import lax
from jax.experimental import pallas as pl
from jax.experimental.pallas import tpu as pltpu
from jax.experimental.pallas import tpu_sc as plsc

D = 32
L = 16
NC = 2
NS = 16
NW = NC * NS
SUB = 128
NBUF = 2
PACK = 128 // D
PD = PACK * D


@functools.lru_cache(maxsize=None)
def _build_score_kernel(total: int, n_main: int, n_rels: int):
    per_w = total // NW
    nsub = per_w // SUB
    n_packed = n_main // PACK
    mesh = plsc.VectorSubcoreMesh(core_axis_name="c", subcore_axis_name="s")

    @functools.partial(
        pl.kernel,
        mesh=mesh,
        compiler_params=pltpu.CompilerParams(needs_layout_passes=False,
                                             use_tc_tiling_on_sc=True),
        out_type=jax.ShapeDtypeStruct((total,), jnp.float32),
        scratch_types=[
            pltpu.VMEM((per_w,), jnp.int32),
            pltpu.VMEM((per_w,), jnp.int32),
            pltpu.VMEM((per_w,), jnp.int32),
            pltpu.VMEM((per_w,), jnp.int32),
            pltpu.VMEM((per_w,), jnp.int32),
            pltpu.VMEM((D,), jnp.float32),
            pltpu.VMEM((n_rels // PACK, PD), jnp.float32),
            pltpu.VMEM((SUB, PD), jnp.float32),
            pltpu.VMEM((SUB, PD), jnp.float32),
            pltpu.VMEM((SUB, PD), jnp.float32),
            pltpu.VMEM((SUB, PD), jnp.float32),
            pltpu.VMEM((per_w,), jnp.float32),
            pltpu.SemaphoreType.DMA,
            pltpu.SemaphoreType.DMA,
        ],
    )
    def score_kernel(head_hbm, tail_hbm, rel_hbm, itab_hbm, rtab_hbm,
                     spare_hbm, out_hbm, hidx, tidx, ridx, hidxp, tidxp,
                     spare_v, rtab_v, hrows0, hrows1, trows0, trows1,
                     acc, sem0, sem1):
        sems = [sem0, sem1]
        hrows = [hrows0, hrows1]
        trows = [trows0, trows1]
        wid = lax.axis_index("s") * NC + lax.axis_index("c")
        base = wid * per_w

        pltpu.sync_copy(head_hbm.at[pl.ds(base, per_w)], hidx)
        pltpu.sync_copy(tail_hbm.at[pl.ds(base, per_w)], tidx)
        pltpu.sync_copy(rel_hbm.at[pl.ds(base, per_w)], ridx)
        pltpu.sync_copy(spare_hbm, spare_v)
        pltpu.sync_copy(rtab_hbm, rtab_v)

        def clamp_body(k, carry):
            hv = jnp.minimum(hidx[pl.ds(k * L, L)], n_main - 1)
            tv = jnp.minimum(tidx[pl.ds(k * L, L)], n_main - 1)
            hidxp[pl.ds(k * L, L)] = lax.shift_right_logical(hv, 2)
            tidxp[pl.ds(k * L, L)] = lax.shift_right_logical(tv, 2)
            return carry

        lax.fori_loop(0, per_w // L, clamp_body, 0)

        def start(c, b):
            off = c * SUB
            pltpu.make_async_copy(itab_hbm.at[hidxp.at[pl.ds(off, SUB)]],
                                  hrows[b], sems[b]).start()
            pltpu.make_async_copy(itab_hbm.at[tidxp.at[pl.ds(off, SUB)]],
                                  trows[b], sems[b]).start()

        def wait(c, b):
            pltpu.make_async_copy(itab_hbm.at[hidxp.at[pl.ds(c * SUB, SUB)]],
                                  hrows[b], sems[b]).wait()
            pltpu.make_async_copy(itab_hbm.at[tidxp.at[pl.ds(c * SUB, SUB)]],
                                  trows[b], sems[b]).wait()

        for b in range(NBUF):
            start(b, b)

        lane = lax.iota(jnp.int32, L)
        three = jnp.full((L,), PACK - 1, jnp.int32)

        def compute(c, b):
            sp0 = spare_v[pl.ds(0, L)]
            sp1 = spare_v[pl.ds(L, L)]

            def group_body(g, carry):
                pos = c * SUB + g * L
                rows = g * L + lane
                hraw = hidx[pl.ds(pos, L)]
                traw = tidx[pl.ds(pos, L)]
                rraw = ridx[pl.ds(pos, L)]
                hoff = lax.shift_left(hraw & three, 5)
                toff = lax.shift_left(traw & three, 5)
                rrow = lax.shift_right_logical(rraw, 2)
                roff = lax.shift_left(rraw & three, 5)
                hm = (hraw >= n_main).astype(jnp.float32)
                tm = (traw >= n_main).astype(jnp.float32)
                s = jnp.zeros((L,), jnp.float32)
                for j in range(D):
                    h = plsc.load_gather(hrows[b], [rows, hoff + j])
                    t = plsc.load_gather(trows[b], [rows, toff + j])
                    r = plsc.load_gather(rtab_v, [rrow, roff + j])
                    spj = sp0[j] if j < L else sp1[j - L]
                    h = h + hm * (spj - h)
                    t = t + tm * (spj - t)
                    d = h + r - t
                    s = s + d * d
                acc[pl.ds(pos, L)] = -s
                return carry

            lax.fori_loop(0, SUB // L, group_body, 0)

        def body(i, carry):
            for b in range(NBUF):
                c = i * NBUF + b
                wait(c, b)
                compute(c, b)
                nxt = c + NBUF

                @pl.when(nxt < nsub)
                def _():
                    start(nxt, b)

            return carry

        lax.fori_loop(0, nsub // NBUF, body, 0)
        pltpu.sync_copy(acc, out_hbm.at[pl.ds(base, per_w)])

    return score_kernel


def kernel(head_ids, tail_ids, relation_ids, i_embeddings, r_embeddings):
    B, K = head_ids.shape
    total = B * K
    n_nodes = i_embeddings.shape[0]
    n_main = (n_nodes // PACK) * PACK
    h1 = head_ids.astype(jnp.int32).reshape(-1)
    t1 = tail_ids.astype(jnp.int32).reshape(-1)
    r1 = relation_ids.astype(jnp.int32).reshape(-1)
    itab_packed = i_embeddings[:n_main].reshape(n_main // PACK, PD)
    rtab_packed = r_embeddings.reshape(r_embeddings.shape[0] // PACK, PD)
    spare = jnp.pad(i_embeddings[n_main:], ((0, 1), (0, 0)))[0]
    score = _build_score_kernel(total, n_main, r_embeddings.shape[0])
    out = score(h1, t1, r1, itab_packed, rtab_packed, spare)
    prediction = out.reshape(B, K)
    pos_pred = prediction[:, :2].reshape(-1)
    neg_pred = prediction[:, 2:].reshape(-1)
    return prediction, pos_pred, neg_pred
